# baseline (speedup 1.0000x reference)
import dataclasses
import functools

import jax
import jax.numpy as jnp
from jax import lax
from jax.experimental import pallas as pl
from jax.experimental.pallas import tpu as pltpu
from jax.experimental.pallas import tpu_sc as plsc

L = 16
NW = 32

D_SP, D_IT, D_AB, D_MV, D_AT = 64, 32, 64, 128, 32
C_SP, C_IT, C_AB, C_MV, C_MA, C_PA = 0, 64, 96, 160, 672, 704
D_OUT = 736


def _build_sc_call(B):
    NC = B // NW
    NB = NC // 2
    assert NB % L == 0

    mesh = plsc.VectorSubcoreMesh(core_axis_name="c", subcore_axis_name="s")
    cp = pltpu.CompilerParams()
    fields_ = pltpu.CompilerParams.__dataclass_fields__
    if "needs_layout_passes" in fields_:
        cp = dataclasses.replace(cp, needs_layout_passes=False)
    if "use_tc_tiling_on_sc" in fields_:
        cp = dataclasses.replace(cp, use_tc_tiling_on_sc=False)

    idx_lists = [pltpu.VMEM((NB,), jnp.int32)] * 7
    piece_bufs = [
        pltpu.VMEM((NB, D_SP), jnp.float32),
        pltpu.VMEM((NB, D_IT), jnp.float32),
        pltpu.VMEM((NB, D_AB), jnp.float32),
        pltpu.VMEM((NB, D_MV), jnp.float32),
        pltpu.VMEM((NB, D_MV), jnp.float32),
        pltpu.VMEM((NB, D_MV), jnp.float32),
        pltpu.VMEM((NB, D_MV), jnp.float32),
        pltpu.VMEM((NB, D_AT), jnp.float32),
        pltpu.VMEM((NB, D_AT), jnp.float32),
    ]

    @functools.partial(
        pl.kernel,
        out_type=[jax.ShapeDtypeStruct((B, D_OUT), jnp.float32)] * 12,
        mesh=mesh,
        compiler_params=cp,
        scratch_types=[
            pltpu.VMEM((NC, 2, 6), jnp.int32),
            pltpu.VMEM((NC, 2, 6), jnp.int32),
            pltpu.VMEM((NC, 2, 6), jnp.int32),
            pltpu.VMEM((NC, 2, 6, 4), jnp.int32),
            *idx_lists, *idx_lists,
            *(piece_bufs * 2),
            pltpu.SemaphoreType.DMA,
            pltpu.SemaphoreType.DMA,
            pltpu.SemaphoreType.DMA,
        ],
    )
    def sc_encoder(sp_hbm, mv_hbm, it_hbm, ab_hbm, ma_hbm, pa_hbm,
                   w_sp, w_mv, w_it, w_ab, *rest):
        outs = rest[:12]
        (sp_blk, it_blk, ab_blk, mv_blk, *more) = rest[12:]
        idxsets = (more[0:7], more[7:14])
        bufsets = (more[14:23], more[23:32])
        rsem = more[32]
        wsems = (more[33], more[34])

        wid = lax.axis_index("s") * 2 + lax.axis_index("c")
        b0 = wid * NC

        for c in [
            pltpu.async_copy(sp_hbm.at[pl.ds(b0, NC)], sp_blk, rsem),
            pltpu.async_copy(it_hbm.at[pl.ds(b0, NC)], it_blk, rsem),
            pltpu.async_copy(ab_hbm.at[pl.ds(b0, NC)], ab_blk, rsem),
            pltpu.async_copy(mv_hbm.at[pl.ds(b0, NC)], mv_blk, rsem),
        ]:
            c.wait()

        iota = lax.iota(jnp.int32, L)
        consts = [jnp.full((L,), v, jnp.int32) for v in range(6)]

        def extract(blk, row0, tail_idx, dst):
            @pl.loop(0, NB // L)
            def _(v):
                rows = iota + (row0 + v * L)
                dst[pl.ds(v * L, L)] = plsc.load_gather(
                    blk, [rows] + [consts[q] for q in tail_idx])

        pending_writes = [None, None]
        items = [(jj, half) for jj in range(12) for half in range(2)]
        for t, (jj, half) in enumerate(items):
            j, i = divmod(jj, 6)
            par = t % 2
            (sp_rows, it_rows, ab_rows, mv0, mv1, mv2, mv3,
             ma_buf, pa_buf) = bufsets[par]
            mv_rows = (mv0, mv1, mv2, mv3)
            ilists = idxsets[par]
            b0c = b0 + half * NB

            if pending_writes[par] is not None:
                for w in pending_writes[par]:
                    w.wait()

            r0 = half * NB
            extract(sp_blk, r0, (j, i), ilists[0])
            extract(it_blk, r0, (j, i), ilists[1])
            extract(ab_blk, r0, (j, i), ilists[2])
            for k in range(4):
                extract(mv_blk, r0, (j, i, k), ilists[3 + k])

            reads = [
                pltpu.async_copy(w_sp.at[ilists[0]], sp_rows, rsem),
                pltpu.async_copy(w_it.at[ilists[1]], it_rows, rsem),
                pltpu.async_copy(w_ab.at[ilists[2]], ab_rows, rsem),
            ]
            reads += [
                pltpu.async_copy(w_mv.at[ilists[3 + k]], mv_rows[k], rsem)
                for k in range(4)
            ]
            reads += [
                pltpu.async_copy(ma_hbm.at[pl.ds(b0c, NB), j, i, q],
                                 ma_buf.at[:, pl.ds(8 * q, 8)], rsem)
                for q in range(4)
            ]
            reads.append(
                pltpu.async_copy(pa_hbm.at[pl.ds(b0c, NB), j, i],
                                 pa_buf, rsem))
            for c in reads:
                c.wait()

            out = outs[jj]
            rows = pl.ds(b0c, NB)
            wsem = wsems[par]
            writes = [
                pltpu.async_copy(sp_rows, out.at[rows, pl.ds(C_SP, D_SP)], wsem),
                pltpu.async_copy(it_rows, out.at[rows, pl.ds(C_IT, D_IT)], wsem),
                pltpu.async_copy(ab_rows, out.at[rows, pl.ds(C_AB, D_AB)], wsem),
            ]
            writes += [
                pltpu.async_copy(mv_rows[k],
                                 out.at[rows, pl.ds(C_MV + k * D_MV, D_MV)],
                                 wsem)
                for k in range(4)
            ]
            writes += [
                pltpu.async_copy(ma_buf, out.at[rows, pl.ds(C_MA, D_AT)], wsem),
                pltpu.async_copy(pa_buf, out.at[rows, pl.ds(C_PA, D_AT)], wsem),
            ]
            pending_writes[par] = writes

        for ws in pending_writes:
            for w in ws:
                w.wait()

    return sc_encoder


def kernel(fields, sides, species, moves, items, abilities, move_attributes,
           pokemon_attributes, W_species, W_moves, W_items, W_abilities):
    B = fields.shape[0]
    sp = species.astype(jnp.int32)
    mv = moves.astype(jnp.int32)
    it = items.astype(jnp.int32)
    ab = abilities.astype(jnp.int32)
    outs = _build_sc_call(B)(sp, mv, it, ab, move_attributes,
                             pokemon_attributes,
                             W_species, W_moves, W_items, W_abilities)
    pokemon_out = tuple(tuple(outs[j * 6 + i] for i in range(6))
                        for j in range(2))
    return (fields, sides, pokemon_out)
